# baseline (speedup 1.0000x reference)
import functools

import jax
import jax.numpy as jnp
from jax import lax
from jax.experimental import pallas as pl
from jax.experimental.pallas import tpu as pltpu
from jax.experimental.pallas import tpu_sc as plsc

N_KNOTS = 21
X_MIN = -5.0
X_MAX = 5.0
IN_FEATURES = 1024
DX = (X_MAX - X_MIN) / (N_KNOTS - 1)
INV_DX = 1.0 / DX

NC = 2
NS = 16
NW = NC * NS
LANES = 16

ROWS_PER_CHUNK = 16
CHUNK = ROWS_PER_CHUNK * IN_FEATURES
VREGS_PER_ROW = IN_FEATURES // LANES


def _spline_body(x_hbm, y_hbm, out_hbm, y_v, a_v, b_v, x_v, o_v,
                 sem_in0, sem_in1, sem_out0, sem_out1, *, n_chunks):
    wid = lax.axis_index("s") * NC + lax.axis_index("c")
    base_row = wid * (n_chunks * ROWS_PER_CHUNK)
    sem_in = (sem_in0, sem_in1)
    sem_out = (sem_out0, sem_out1)

    pltpu.sync_copy(y_hbm, y_v.at[pl.ds(0, IN_FEATURES * N_KNOTS)])

    iota = lax.iota(jnp.int32, LANES)
    iota21 = iota * N_KNOTS

    @plsc.parallel_loop(0, (IN_FEATURES * N_KNOTS) // LANES, unroll=4)
    def _(v):
        n0 = v * LANES
        sl = pl.ds(n0, LANES)
        nv = iota + n0
        yl = y_v[sl]
        yr = plsc.load_gather(y_v, [nv + 1])
        k = jnp.remainder(nv, N_KNOTS)
        knot = X_MIN + DX * k.astype(jnp.float32)
        bv = jnp.where(k == N_KNOTS - 1, 0.0, (yr - yl) * INV_DX)
        a_v[sl] = yl - bv * knot
        b_v[sl] = bv

    def fire_in(b, c):
        r0 = base_row + c * ROWS_PER_CHUNK
        pltpu.async_copy(x_hbm.at[pl.ds(r0, ROWS_PER_CHUNK)],
                         x_v.at[b], sem_in[b])

    def wait_in(b):
        pltpu.make_async_copy(x_hbm.at[pl.ds(base_row, ROWS_PER_CHUNK)],
                              x_v.at[b], sem_in[b]).wait()

    def fire_out(b, c):
        r0 = base_row + c * ROWS_PER_CHUNK
        pltpu.async_copy(o_v.at[b],
                         out_hbm.at[pl.ds(r0, ROWS_PER_CHUNK)],
                         sem_out[b])

    def wait_out(b):
        pltpu.make_async_copy(o_v.at[b],
                              out_hbm.at[pl.ds(base_row, ROWS_PER_CHUNK)],
                              sem_out[b]).wait()

    def compute(b):
        @plsc.parallel_loop(0, CHUNK // LANES, unroll=16)
        def _(v):
            r = v >> 6
            j = v & (VREGS_PER_ROW - 1)
            sl = pl.ds(j * LANES, LANES)
            fb = j * (LANES * N_KNOTS)
            xv = x_v[b, r, sl]
            xc = jnp.minimum(jnp.maximum(xv, X_MIN), X_MAX)
            pos = (xc - X_MIN) * INV_DX
            flat = pos.astype(jnp.int32) + iota21
            tile = pl.ds(fb, LANES * N_KNOTS)
            av = plsc.load_gather(a_v.at[tile], [flat])
            bv = plsc.load_gather(b_v.at[tile], [flat])
            o_v[b, r, sl] = av + bv * xc

    fire_in(0, 0)
    fire_in(1, 1)

    def pair_body(g, carry):
        for b in range(2):
            c = 2 * g + b
            wait_in(b)
            pl.when(c >= 2)(lambda: wait_out(b))
            compute(b)
            fire_out(b, c)
            pl.when(c + 2 < n_chunks)(lambda: fire_in(b, c + 2))
        return carry

    lax.fori_loop(0, n_chunks // 2, pair_body, 0)
    wait_out(0)
    wait_out(1)


def kernel(x, y):
    orig_shape = x.shape
    n = x.size
    n_rows = n // IN_FEATURES
    assert n % (NW * 2 * CHUNK) == 0
    n_chunks = n // (NW * CHUNK)

    x2 = x.reshape(n_rows, IN_FEATURES)
    mesh = plsc.VectorSubcoreMesh(core_axis_name="c", subcore_axis_name="s")
    run = pl.kernel(
        functools.partial(_spline_body, n_chunks=n_chunks),
        out_type=jax.ShapeDtypeStruct((n_rows, IN_FEATURES), jnp.float32),
        mesh=mesh,
        compiler_params=pltpu.CompilerParams(needs_layout_passes=False),
        scratch_types=[
            pltpu.VMEM((IN_FEATURES * N_KNOTS + LANES,), jnp.float32),
            pltpu.VMEM((IN_FEATURES * N_KNOTS,), jnp.float32),
            pltpu.VMEM((IN_FEATURES * N_KNOTS,), jnp.float32),
            pltpu.VMEM((2, ROWS_PER_CHUNK, IN_FEATURES), jnp.float32),
            pltpu.VMEM((2, ROWS_PER_CHUNK, IN_FEATURES), jnp.float32),
            pltpu.SemaphoreType.DMA,
            pltpu.SemaphoreType.DMA,
            pltpu.SemaphoreType.DMA,
            pltpu.SemaphoreType.DMA,
        ],
    )
    out2 = run(x2, y.reshape(IN_FEATURES * N_KNOTS))
    return out2.reshape(orig_shape)


# speedup vs baseline: 1.2159x; 1.2159x over previous
import functools

import jax
import jax.numpy as jnp
from jax import lax
from jax.experimental import pallas as pl
from jax.experimental.pallas import tpu as pltpu
from jax.experimental.pallas import tpu_sc as plsc

N_KNOTS = 21
X_MIN = -5.0
X_MAX = 5.0
IN_FEATURES = 1024
DX = (X_MAX - X_MIN) / (N_KNOTS - 1)
INV_DX = 1.0 / DX

NC = 2
NS = 16
NW = NC * NS
LANES = 16

ROWS_PER_CHUNK = 16
CHUNK = ROWS_PER_CHUNK * IN_FEATURES
VREGS_PER_ROW = IN_FEATURES // LANES


def _spline_body(x_hbm, y_hbm, out_hbm, y_v, a_v, b_v, x_v, o_v,
                 sem_in0, sem_in1, sem_out0, sem_out1, sem_y, *, n_chunks):
    wid = lax.axis_index("s") * NC + lax.axis_index("c")
    base_row = wid * (n_chunks * ROWS_PER_CHUNK)
    sem_in = (sem_in0, sem_in1)
    sem_out = (sem_out0, sem_out1)

    y_copy = pltpu.async_copy(y_hbm, y_v.at[pl.ds(0, IN_FEATURES * N_KNOTS)],
                              sem_y)

    iota = lax.iota(jnp.int32, LANES)
    iota21 = iota * N_KNOTS

    def build_tables():
        @plsc.parallel_loop(0, (IN_FEATURES * N_KNOTS) // LANES, unroll=8)
        def _(v):
            n0 = v * LANES
            sl = pl.ds(n0, LANES)
            nv = iota + n0
            yl = y_v[sl]
            yr = plsc.load_gather(y_v, [nv + 1])
            k = jnp.remainder(nv, N_KNOTS)
            knot = X_MIN + DX * k.astype(jnp.float32)
            bv = jnp.where(k == N_KNOTS - 1, 0.0, (yr - yl) * INV_DX)
            a_v[sl] = yl - bv * knot
            b_v[sl] = bv

    def fire_in(b, c):
        r0 = base_row + c * ROWS_PER_CHUNK
        pltpu.async_copy(x_hbm.at[pl.ds(r0, ROWS_PER_CHUNK)],
                         x_v.at[b], sem_in[b])

    def wait_in(b):
        pltpu.make_async_copy(x_hbm.at[pl.ds(base_row, ROWS_PER_CHUNK)],
                              x_v.at[b], sem_in[b]).wait()

    def fire_out(b, c):
        r0 = base_row + c * ROWS_PER_CHUNK
        pltpu.async_copy(o_v.at[b],
                         out_hbm.at[pl.ds(r0, ROWS_PER_CHUNK)],
                         sem_out[b])

    def wait_out(b):
        pltpu.make_async_copy(o_v.at[b],
                              out_hbm.at[pl.ds(base_row, ROWS_PER_CHUNK)],
                              sem_out[b]).wait()

    def compute(b):
        @plsc.parallel_loop(0, CHUNK // LANES, unroll=8)
        def _(v):
            r = v >> 6
            j = v & (VREGS_PER_ROW - 1)
            sl = pl.ds(j * LANES, LANES)
            fb = j * (LANES * N_KNOTS)
            xv = x_v[b, r, sl]
            xc = jnp.minimum(jnp.maximum(xv, X_MIN), X_MAX)
            pos = (xc - X_MIN) * INV_DX
            flat = pos.astype(jnp.int32) + iota21
            tile = pl.ds(fb, LANES * N_KNOTS)
            av = plsc.load_gather(a_v.at[tile], [flat])
            bv = plsc.load_gather(b_v.at[tile], [flat])
            o_v[b, r, sl] = av + bv * xc

    fire_in(0, 0)
    fire_in(1, 1)
    y_copy.wait()
    build_tables()

    def pair_body(g, carry):
        for b in range(2):
            c = 2 * g + b
            wait_in(b)
            pl.when(c >= 2)(lambda: wait_out(b))
            compute(b)
            fire_out(b, c)
            pl.when(c + 2 < n_chunks)(lambda: fire_in(b, c + 2))
        return carry

    lax.fori_loop(0, n_chunks // 2, pair_body, 0)
    wait_out(0)
    wait_out(1)


def kernel(x, y):
    orig_shape = x.shape
    n = x.size
    n_rows = n // IN_FEATURES
    assert n % (NW * 2 * CHUNK) == 0
    n_chunks = n // (NW * CHUNK)

    x2 = x.reshape(n_rows, IN_FEATURES)
    mesh = plsc.VectorSubcoreMesh(core_axis_name="c", subcore_axis_name="s")
    run = pl.kernel(
        functools.partial(_spline_body, n_chunks=n_chunks),
        out_type=jax.ShapeDtypeStruct((n_rows, IN_FEATURES), jnp.float32),
        mesh=mesh,
        compiler_params=pltpu.CompilerParams(needs_layout_passes=False),
        scratch_types=[
            pltpu.VMEM((IN_FEATURES * N_KNOTS + LANES,), jnp.float32),
            pltpu.VMEM((IN_FEATURES * N_KNOTS,), jnp.float32),
            pltpu.VMEM((IN_FEATURES * N_KNOTS,), jnp.float32),
            pltpu.VMEM((2, ROWS_PER_CHUNK, IN_FEATURES), jnp.float32),
            pltpu.VMEM((2, ROWS_PER_CHUNK, IN_FEATURES), jnp.float32),
            pltpu.SemaphoreType.DMA,
            pltpu.SemaphoreType.DMA,
            pltpu.SemaphoreType.DMA,
            pltpu.SemaphoreType.DMA,
            pltpu.SemaphoreType.DMA,
        ],
    )
    out2 = run(x2, y.reshape(IN_FEATURES * N_KNOTS))
    return out2.reshape(orig_shape)
